# baseline (speedup 1.0000x reference)
import functools

import jax
import jax.numpy as jnp
from jax import lax
from jax.experimental import pallas as pl
from jax.experimental.pallas import tpu as pltpu
from jax.experimental.pallas import tpu_sc as plsc

_NUM_FIELDS = 26
_VOCAB = 100000
_EMB = 64
_BATCH = 4096
_ROWS = _NUM_FIELDS * _EMB
_NW = 32
_PER_W = _ROWS // _NW
_LANES = 16
_NSLICE = _BATCH // _LANES
_H0 = 50048
_H1 = _VOCAB - _H0
_PAD = 2 * _LANES
_LCAP = _BATCH + _PAD

_mesh = plsc.VectorSubcoreMesh(core_axis_name="c", subcore_axis_name="s")


@functools.partial(
    pl.kernel,
    mesh=_mesh,
    out_type=jax.ShapeDtypeStruct((_ROWS, _BATCH), jnp.float32),
    scratch_types=[
        pltpu.VMEM((_H0,), jnp.float32),
        pltpu.VMEM((_H1,), jnp.float32),
        pltpu.VMEM((_BATCH,), jnp.int32),
        pltpu.VMEM((_LCAP,), jnp.int32),
        pltpu.VMEM((_LCAP,), jnp.int32),
        pltpu.VMEM((_LCAP,), jnp.int32),
        pltpu.VMEM((_LCAP,), jnp.int32),
        pltpu.VMEM((_BATCH + _PAD,), jnp.float32),
        pltpu.SemaphoreType.DMA,
        pltpu.SemaphoreType.DMA,
        pltpu.SemaphoreType.DMA,
    ],
    compiler_params=pltpu.CompilerParams(
        use_tc_tiling_on_sc=True, needs_layout_passes=False
    ),
)
def _emb_rows(
    x_t, table, out_t, b0, b1, idx_v, o0, q0, o1, q1, val_v, s0, s1, wsem
):
    wid = lax.axis_index("s") * 2 + lax.axis_index("c")
    r0 = wid * _PER_W
    lanes = lax.iota(jnp.int32, _LANES)

    def do_span(f, lo, hi):
        pltpu.sync_copy(x_t.at[f], idx_v)

        def pbody(t, ptrs):
            p0, p1 = ptrs
            iv = idx_v[pl.ds(t * _LANES, _LANES)]
            pos = t * _LANES + lanes
            m0 = iv < _H0
            m1 = ~m0
            plsc.store_compressed(o0.at[pl.ds(p0, _LANES)], iv, mask=m0)
            plsc.store_compressed(q0.at[pl.ds(p0, _LANES)], pos, mask=m0)
            plsc.store_compressed(o1.at[pl.ds(p1, _LANES)], iv - _H0, mask=m1)
            plsc.store_compressed(q1.at[pl.ds(p1, _LANES)], pos, mask=m1)
            c0 = jnp.sum(jnp.where(m0, 1, 0))
            return p0 + c0, p1 + (_LANES - c0)

        n0, n1 = lax.fori_loop(0, _NSLICE, pbody, (0, 0))
        zero16 = jnp.zeros((_LANES,), jnp.int32)
        for u in range(2):
            slack = _BATCH + u * _LANES + lanes
            o0[pl.ds(n0 + u * _LANES, _LANES)] = zero16
            q0[pl.ds(n0 + u * _LANES, _LANES)] = slack
            o1[pl.ds(n1 + u * _LANES, _LANES)] = zero16
            q1[pl.ds(n1 + u * _LANES, _LANES)] = slack

        def issue0(r):
            return pltpu.async_copy(table.at[r, pl.ds(0, _H0)], b0, s0)

        def issue1(r):
            return pltpu.async_copy(table.at[r, pl.ds(_H0, _H1)], b1, s1)

        issue0(lo)
        issue1(lo)

        def gpass(buf, offs, poss, n):
            def gbody(t, _):
                for u in range(2):
                    sl = pl.ds((2 * t + u) * _LANES, _LANES)
                    g = plsc.load_gather(buf, [offs[sl]])
                    plsc.store_scatter(val_v, [poss[sl]], g)
                return 0

            lax.fori_loop(0, (n + _PAD - 1) // _PAD, gbody, 0)

        def row_body(r, _):
            pltpu.make_async_copy(table.at[r, pl.ds(0, _H0)], b0, s0).wait()

            @pl.when(r > r0)
            def _():
                pltpu.make_async_copy(
                    val_v.at[pl.ds(0, _BATCH)], out_t.at[r - 1], wsem
                ).wait()

            gpass(b0, o0, q0, n0)

            @pl.when(r + 1 < hi)
            def _():
                issue0(r + 1)

            pltpu.make_async_copy(table.at[r, pl.ds(_H0, _H1)], b1, s1).wait()
            gpass(b1, o1, q1, n1)

            @pl.when(r + 1 < hi)
            def _():
                issue1(r + 1)

            pltpu.async_copy(val_v.at[pl.ds(0, _BATCH)], out_t.at[r], wsem)
            return 0

        lax.fori_loop(lo, hi, row_body, 0)

    f0 = r0 // _EMB
    mid = jnp.minimum(r0 + _PER_W, (f0 + 1) * _EMB)
    do_span(f0, r0, mid)

    @pl.when(mid < r0 + _PER_W)
    def _():
        do_span(f0 + 1, mid, r0 + _PER_W)

    pltpu.make_async_copy(
        val_v.at[pl.ds(0, _BATCH)], out_t.at[r0 + _PER_W - 1], wsem
    ).wait()


def kernel(x, tables):
    table2d = jnp.swapaxes(tables, 1, 2).reshape(_ROWS, _VOCAB)
    x_t = x.T
    out_t = _emb_rows(x_t, table2d)
    return out_t.T.reshape(_BATCH, _ROWS)
